# baseline (speedup 1.0000x reference)
import functools

import jax
import jax.numpy as jnp
from jax import lax
from jax.experimental import pallas as pl
from jax.experimental.pallas import tpu as pltpu
from jax.experimental.pallas import tpu_sc as plsc

N = 10000
E = 320000
EMB = 128
NL = 5
B = 64
NC = 10


NSC = 2
NTILE = 16
NW = NSC * NTILE
EP = E // NW
CW = 80
NCHUNK = EP // CW
NP = 10240
RPT = NP // NTILE

SCW = 120
NSLOT = 3
EPP = 10080
EPADT = NW * EPP
NS = EPP // SCW
NS0 = 66
NS1 = 2 * NS - NS0
L0 = NS0 * SCW
L1 = NS1 * SCW


def _sc_edge_body(hl, srcr, dstr, ea, out, accum,
                  si0, si1, si2, di0, di1, di2,
                  b0, b1, b2,
                  is0, is1, is2, gs0, gs1, gs2,
                  ss0, ss1, ss2):
  si = (si0, si1, si2)
  di = (di0, di1, di2)
  buf = (b0, b1, b2)
  isem = (is0, is1, is2)
  gsem = (gs0, gs1, gs2)
  ssem = (ss0, ss1, ss2)

  c = lax.axis_index("c")
  s = lax.axis_index("s")
  wid = c * NTILE + s
  r0 = s * RPT
  zero16 = jnp.zeros((16,), jnp.float32)

  def zrow(r, c2):
    for j in range(8):
      buf[0][r, pl.ds(j * 16, 16)] = zero16
    return c2

  lax.fori_loop(0, SCW, zrow, 0)
  for k in range(RPT // SCW):
    pltpu.sync_copy(buf[0], accum.at[pl.ds(r0 + k * SCW, SCW)])
  _rem = RPT - (RPT // SCW) * SCW
  if _rem:
    pltpu.sync_copy(buf[0].at[pl.ds(0, _rem)],
                    accum.at[pl.ds(r0 + (RPT // SCW) * SCW, _rem)])
  plsc.subcore_barrier()

  base = jnp.where(c == 0, s * L0, NTILE * L0 + s * L1)
  ns_me = jnp.where(c == 0, NS0, NS1)

  def fire_in(sv, b):
    off = pl.multiple_of(base + sv * SCW, 8)
    offe = pl.multiple_of(jnp.minimum(off, E - SCW), 8)
    pltpu.async_copy(srcr.at[pl.ds(off, SCW)], si[b], isem[b])
    pltpu.async_copy(dstr.at[pl.ds(off, SCW)], di[b], isem[b])
    pltpu.async_copy(ea.at[pl.ds(offe, SCW)], buf[b], isem[b])

  def wait_in(b):
    pltpu.make_async_copy(srcr.at[pl.ds(0, SCW)], si[b], isem[b]).wait()
    pltpu.make_async_copy(dstr.at[pl.ds(0, SCW)], di[b], isem[b]).wait()
    pltpu.make_async_copy(ea.at[pl.ds(0, SCW)], buf[b], isem[b]).wait()

  def fire_gather(b):
    pltpu.async_copy(hl.at[si[b]], buf[b], gsem[b], add=True)

  def wait_gather(b):
    pltpu.make_async_copy(hl.at[si[b]], buf[b], gsem[b]).wait()

  def fire_scatter(b):
    pltpu.async_copy(buf[b], accum.at[di[b]], ssem[b], add=True)

  def wait_scatter(b):
    pltpu.make_async_copy(buf[b], accum.at[di[b]], ssem[b]).wait()

  def relu_buf(b):
    def rrow(r, c2):
      for j in range(8):
        sl = pl.ds(j * 16, 16)
        buf[b][r, sl] = jnp.maximum(buf[b][r, sl], 0.0)
      return c2

    lax.fori_loop(0, SCW, rrow, 0, unroll=4)

  fire_in(0, 0)
  fire_in(1, 1)
  wait_in(0)
  fire_gather(0)

  def body(i, carry):
    for b in range(NSLOT):
      sv = NSLOT * i + b
      wait_gather(b)
      b1 = (b + 1) % NSLOT

      @pl.when(sv + 1 < ns_me)
      def _():
        wait_in(b1)
        fire_gather(b1)

      relu_buf(b)
      fire_scatter(b)
      b2 = (b + 2) % NSLOT

      @pl.when((sv >= 1) & (sv + 2 < ns_me))
      def _():
        wait_scatter(b2)
        fire_in(sv + 2, b2)

      @pl.when((sv < 1) & (sv + 2 < ns_me))
      def _():
        fire_in(sv + 2, b2)

    return carry

  lax.fori_loop(0, ns_me // NSLOT, body, 0)

  for b in range(NSLOT):
    wait_scatter(b)
  plsc.subcore_barrier()
  pltpu.sync_copy(accum.at[pl.ds(r0, RPT)], out.at[pl.ds(c * NP + r0, RPT)])


@functools.cache
def _sc_edge_call():
  mesh = plsc.VectorSubcoreMesh(
      core_axis_name="c", subcore_axis_name="s",
      num_cores=NSC, num_subcores=NTILE)
  return pl.kernel(
      _sc_edge_body,
      out_type=jax.ShapeDtypeStruct((NSC * NP, EMB), jnp.float32),
      mesh=mesh,
      scratch_types=(
          [pltpu.VMEM_SHARED((NP, EMB), jnp.float32)]
          + [pltpu.VMEM((SCW,), jnp.int32) for _ in range(NSLOT)]
          + [pltpu.VMEM((SCW,), jnp.int32) for _ in range(NSLOT)]
          + [pltpu.VMEM((SCW, EMB), jnp.float32) for _ in range(NSLOT)]
          + [pltpu.SemaphoreType.DMA for _ in range(3 * NSLOT)]
      ),
  )


def _sc_edge_pass(hl, srcr, dstr, ea):
  return _sc_edge_call()(hl, srcr, dstr, ea)


BLK = 2000
GRID = N // BLK


def _onehot(b_ref):
  return (b_ref[...] == lax.broadcasted_iota(jnp.int32, (BLK, B), 1)
          ).astype(jnp.float32)


def _segsum(oh, x):
  return lax.dot_general(oh, x, (((0,), (0,)), ((), ())),
                         preferred_element_type=jnp.float32)


def _vn_mlp(vt, w1_ref, b1_ref, w2_ref, b2_ref):
  u = jnp.maximum(
      jnp.dot(vt, w1_ref[...], preferred_element_type=jnp.float32)
      + b1_ref[...], 0.0)
  return jnp.maximum(
      jnp.dot(u, w2_ref[...], preferred_element_type=jnp.float32)
      + b2_ref[...], 0.0)


def _init_body(x_ref, b_ref, vne_ref, w1_ref, b1_ref, w2_ref, b2_ref,
               hl_ref, vn1_ref, seg_ref):
  i = pl.program_id(0)
  hl = x_ref[...] + vne_ref[...]
  hl_ref[...] = hl
  part = _segsum(_onehot(b_ref), hl)

  @pl.when(i == 0)
  def _():
    seg_ref[...] = part

  @pl.when(i > 0)
  def _():
    seg_ref[...] += part

  @pl.when(i == GRID - 1)
  def _():
    vt = seg_ref[...] + vne_ref[...]
    vn1_ref[...] = _vn_mlp(vt, w1_ref, b1_ref, w2_ref, b2_ref)


def _mid_body(hl_ref, aa_ref, ab_ref, b_ref, vn_ref, eps_ref,
              w1_ref, b1_ref, w2_ref, b2_ref,
              vw1_ref, vb1_ref, vw2_ref, vb2_ref,
              out_ref, vn2_ref, seg_ref):
  i = pl.program_id(0)
  z2 = eps_ref[0, 0] * hl_ref[...] + aa_ref[...] + ab_ref[...]
  u = jnp.maximum(
      jnp.dot(z2, w1_ref[...], preferred_element_type=jnp.float32)
      + b1_ref[...], 0.0)
  z = jnp.dot(u, w2_ref[...], preferred_element_type=jnp.float32) + b2_ref[...]
  h = jnp.maximum(z, 0.0)
  oh = _onehot(b_ref)
  hln = h + jnp.dot(oh, vn_ref[...], preferred_element_type=jnp.float32)
  out_ref[...] = hln
  part = _segsum(oh, hln)

  @pl.when(i == 0)
  def _():
    seg_ref[...] = part

  @pl.when(i > 0)
  def _():
    seg_ref[...] += part

  @pl.when(i == GRID - 1)
  def _():
    vt = seg_ref[...] + vn_ref[...]
    vn2_ref[...] = _vn_mlp(vt, vw1_ref, vb1_ref, vw2_ref, vb2_ref)


def _last_mid_body(hl_ref, aa_ref, ab_ref, b_ref, vn_ref, eps_ref,
                   w1_ref, b1_ref, w2_ref, b2_ref, out_ref):
  z2 = eps_ref[0, 0] * hl_ref[...] + aa_ref[...] + ab_ref[...]
  u = jnp.maximum(
      jnp.dot(z2, w1_ref[...], preferred_element_type=jnp.float32)
      + b1_ref[...], 0.0)
  z = jnp.dot(u, w2_ref[...], preferred_element_type=jnp.float32) + b2_ref[...]
  h = jnp.maximum(z, 0.0)
  oh = _onehot(b_ref)
  out_ref[...] = h + jnp.dot(oh, vn_ref[...],
                             preferred_element_type=jnp.float32)


def _final_body(hl_ref, aa_ref, ab_ref, b_ref, eps_ref,
                w1_ref, b1_ref, w2_ref, b2_ref, hw_ref, hb_ref,
                out_ref, pool_ref, cnt_ref):
  i = pl.program_id(0)
  z2 = eps_ref[0, 0] * hl_ref[...] + aa_ref[...] + ab_ref[...]
  u = jnp.maximum(
      jnp.dot(z2, w1_ref[...], preferred_element_type=jnp.float32)
      + b1_ref[...], 0.0)
  z = jnp.dot(u, w2_ref[...], preferred_element_type=jnp.float32) + b2_ref[...]
  oh = _onehot(b_ref)
  part = _segsum(oh, z)
  pcnt = _segsum(oh, jnp.ones((BLK, EMB), jnp.float32))

  @pl.when(i == 0)
  def _():
    pool_ref[...] = part
    cnt_ref[...] = pcnt

  @pl.when(i > 0)
  def _():
    pool_ref[...] += part
    cnt_ref[...] += pcnt

  @pl.when(i == GRID - 1)
  def _():
    hg = pool_ref[...] / jnp.maximum(cnt_ref[...], 1.0)
    out_ref[...] = (jnp.dot(hg, hw_ref[...],
                            preferred_element_type=jnp.float32)
                    + hb_ref[...])


def _blk(i):
  return (i, 0)


def _full(i):
  return (0, 0)


_ROW = pl.BlockSpec((BLK, EMB), _blk)
_ROWB = pl.BlockSpec((BLK, 1), _blk)


def _fullspec(shape):
  return pl.BlockSpec(shape, _full)


def _tc_init(x, batch2, vne, w1, b1, w2, b2):
  return pl.pallas_call(
      _init_body,
      grid=(GRID,),
      in_specs=[
          _ROW, _ROWB, _fullspec((1, EMB)),
          _fullspec((EMB, 2 * EMB)), _fullspec((1, 2 * EMB)),
          _fullspec((2 * EMB, EMB)), _fullspec((1, EMB)),
      ],
      out_specs=[_ROW, _fullspec((B, EMB))],
      out_shape=[
          jax.ShapeDtypeStruct((N, EMB), jnp.float32),
          jax.ShapeDtypeStruct((B, EMB), jnp.float32),
      ],
      scratch_shapes=[pltpu.VMEM((B, EMB), jnp.float32)],
  )(x, batch2, vne, w1, b1, w2, b2)


def _tc_mid(hl, aa, ab, batch2, vn, eps, w1, b1, w2, b2, vw1, vb1, vw2, vb2):
  return pl.pallas_call(
      _mid_body,
      grid=(GRID,),
      in_specs=[
          _ROW, _ROW, _ROW, _ROWB, _fullspec((B, EMB)), _fullspec((1, 1)),
          _fullspec((EMB, 2 * EMB)), _fullspec((1, 2 * EMB)),
          _fullspec((2 * EMB, EMB)), _fullspec((1, EMB)),
          _fullspec((EMB, 2 * EMB)), _fullspec((1, 2 * EMB)),
          _fullspec((2 * EMB, EMB)), _fullspec((1, EMB)),
      ],
      out_specs=[_ROW, _fullspec((B, EMB))],
      out_shape=[
          jax.ShapeDtypeStruct((N, EMB), jnp.float32),
          jax.ShapeDtypeStruct((B, EMB), jnp.float32),
      ],
      scratch_shapes=[pltpu.VMEM((B, EMB), jnp.float32)],
  )(hl, aa, ab, batch2, vn, eps, w1, b1, w2, b2, vw1, vb1, vw2, vb2)


def _tc_last_mid(hl, aa, ab, batch2, vn, eps, w1, b1, w2, b2):
  return pl.pallas_call(
      _last_mid_body,
      grid=(GRID,),
      in_specs=[
          _ROW, _ROW, _ROW, _ROWB, _fullspec((B, EMB)), _fullspec((1, 1)),
          _fullspec((EMB, 2 * EMB)), _fullspec((1, 2 * EMB)),
          _fullspec((2 * EMB, EMB)), _fullspec((1, EMB)),
      ],
      out_specs=_ROW,
      out_shape=jax.ShapeDtypeStruct((N, EMB), jnp.float32),
  )(hl, aa, ab, batch2, vn, eps, w1, b1, w2, b2)


def _tc_final(hl, aa, ab, batch2, eps, w1, b1, w2, b2, hw, hb):
  return pl.pallas_call(
      _final_body,
      grid=(GRID,),
      in_specs=[
          _ROW, _ROW, _ROW, _ROWB, _fullspec((1, 1)),
          _fullspec((EMB, 2 * EMB)), _fullspec((1, 2 * EMB)),
          _fullspec((2 * EMB, EMB)), _fullspec((1, EMB)),
          _fullspec((EMB, NC)), _fullspec((1, NC)),
      ],
      out_specs=_fullspec((B, NC)),
      out_shape=jax.ShapeDtypeStruct((B, NC), jnp.float32),
      scratch_shapes=[
          pltpu.VMEM((B, EMB), jnp.float32),
          pltpu.VMEM((B, EMB), jnp.float32),
      ],
  )(hl, aa, ab, batch2, eps, w1, b1, w2, b2, hw, hb)


_INV = 1.0 / jnp.sqrt(1.0 + 1e-5)


def kernel(x, edge_index, edge_attr, batch, gin_W1, gin_b1, gin_bn1_g,
           gin_bn1_b, gin_W2, gin_b2, eps, bn_g, bn_b, vn_emb, vn_W1, vn_b1,
           vn_bn1_g, vn_bn1_b, vn_W2, vn_b2, vn_bn2_g, vn_bn2_b, head_W,
           head_b):
  src = jnp.concatenate(
      [edge_index[0], jnp.zeros((EPADT - E,), jnp.int32)])
  dst = jnp.concatenate(
      [edge_index[1], jnp.full((EPADT - E,), NP - 1, jnp.int32)])
  s1 = gin_bn1_g * _INV
  w1f = gin_W1 * s1[:, None, :]
  b1f = (gin_b1 * s1 + gin_bn1_b)[:, None, :]
  s2 = bn_g * _INV
  w2f = gin_W2 * s2[:, None, :]
  b2f = (gin_b2 * s2 + bn_b)[:, None, :]
  t1 = vn_bn1_g * _INV
  vw1f = vn_W1 * t1[:, None, :]
  vb1f = (vn_b1 * t1 + vn_bn1_b)[:, None, :]
  t2 = vn_bn2_g * _INV
  vw2f = vn_W2 * t2[:, None, :]
  vb2f = (vn_b2 * t2 + vn_bn2_b)[:, None, :]
  eps1 = (1.0 + eps).reshape(NL, 1, 1)
  batch2 = batch.reshape(N, 1)
  vne = vn_emb.reshape(1, EMB)
  hb2 = head_b.reshape(1, NC)

  hl, vn = _tc_init(x, batch2, vne, vw1f[0], vb1f[0], vw2f[0], vb2f[0])
  for l in range(NL - 1):
    part = _sc_edge_pass(hl, src, dst, edge_attr)
    aa = part[:N]
    ab = part[NP:NP + N]
    if l < NL - 2:
      hl, vn = _tc_mid(hl, aa, ab, batch2, vn, eps1[l],
                       w1f[l], b1f[l], w2f[l], b2f[l],
                       vw1f[l + 1], vb1f[l + 1], vw2f[l + 1], vb2f[l + 1])
    else:
      hl = _tc_last_mid(hl, aa, ab, batch2, vn, eps1[l],
                        w1f[l], b1f[l], w2f[l], b2f[l])
  part = _sc_edge_pass(hl, src, dst, edge_attr)
  return _tc_final(hl, part[:N], part[NP:NP + N], batch2, eps1[NL - 1],
                   w1f[NL - 1], b1f[NL - 1], w2f[NL - 1], b2f[NL - 1],
                   head_W, hb2)


# speedup vs baseline: 1.1994x; 1.1994x over previous
import functools

import jax
import jax.numpy as jnp
from jax import lax
from jax.experimental import pallas as pl
from jax.experimental.pallas import tpu as pltpu
from jax.experimental.pallas import tpu_sc as plsc

N = 10000
E = 320000
EMB = 128
NL = 5
B = 64
NC = 10


NSC = 2
NTILE = 16
NW = NSC * NTILE
EP = E // NW
CW = 80
NCHUNK = EP // CW
NP = 10240
RPT = NP // NTILE

SCW = 120
NSLOT = 3
EPP = 10080
EPADT = NW * EPP
NS = EPP // SCW
NS0 = 102
NS1 = 2 * NS - NS0
L0 = NS0 * SCW
L1 = NS1 * SCW


def _sc_edge_body(hl, srcr, dstr, ea, out, accum,
                  si0, si1, si2, di0, di1, di2,
                  b0, b1, b2,
                  is0, is1, is2, gs0, gs1, gs2,
                  ss0, ss1, ss2):
  si = (si0, si1, si2)
  di = (di0, di1, di2)
  buf = (b0, b1, b2)
  isem = (is0, is1, is2)
  gsem = (gs0, gs1, gs2)
  ssem = (ss0, ss1, ss2)

  c = lax.axis_index("c")
  s = lax.axis_index("s")
  wid = c * NTILE + s
  r0 = s * RPT
  zero16 = jnp.zeros((16,), jnp.float32)

  def zrow(r, c2):
    for j in range(8):
      buf[0][r, pl.ds(j * 16, 16)] = zero16
    return c2

  lax.fori_loop(0, SCW, zrow, 0)
  for k in range(RPT // SCW):
    pltpu.sync_copy(buf[0], accum.at[pl.ds(r0 + k * SCW, SCW)])
  _rem = RPT - (RPT // SCW) * SCW
  if _rem:
    pltpu.sync_copy(buf[0].at[pl.ds(0, _rem)],
                    accum.at[pl.ds(r0 + (RPT // SCW) * SCW, _rem)])
  plsc.subcore_barrier()

  base = jnp.where(c == 0, s * L0, NTILE * L0 + s * L1)
  ns_me = jnp.where(c == 0, NS0, NS1)

  def fire_in(sv, b):
    off = pl.multiple_of(base + sv * SCW, 8)
    offe = pl.multiple_of(jnp.minimum(off, E - SCW), 8)
    pltpu.async_copy(srcr.at[pl.ds(off, SCW)], si[b], isem[b])
    pltpu.async_copy(dstr.at[pl.ds(off, SCW)], di[b], isem[b])
    pltpu.async_copy(ea.at[pl.ds(offe, SCW)], buf[b], isem[b])

  def wait_in(b):
    pltpu.make_async_copy(srcr.at[pl.ds(0, SCW)], si[b], isem[b]).wait()
    pltpu.make_async_copy(dstr.at[pl.ds(0, SCW)], di[b], isem[b]).wait()
    pltpu.make_async_copy(ea.at[pl.ds(0, SCW)], buf[b], isem[b]).wait()

  def fire_gather(b):
    pltpu.async_copy(hl.at[si[b]], buf[b], gsem[b], add=True)

  def wait_gather(b):
    pltpu.make_async_copy(hl.at[si[b]], buf[b], gsem[b]).wait()

  def fire_scatter(b):
    pltpu.async_copy(buf[b], accum.at[di[b]], ssem[b], add=True)

  def wait_scatter(b):
    pltpu.make_async_copy(buf[b], accum.at[di[b]], ssem[b]).wait()

  def relu_buf(b):
    def rrow(r, c2):
      for j in range(8):
        sl = pl.ds(j * 16, 16)
        buf[b][r, sl] = jnp.maximum(buf[b][r, sl], 0.0)
      return c2

    lax.fori_loop(0, SCW, rrow, 0, unroll=4)

  fire_in(0, 0)
  fire_in(1, 1)
  wait_in(0)
  fire_gather(0)

  def body(i, carry):
    for b in range(NSLOT):
      sv = NSLOT * i + b
      wait_gather(b)
      b1 = (b + 1) % NSLOT

      @pl.when(sv + 1 < ns_me)
      def _():
        wait_in(b1)
        fire_gather(b1)

      relu_buf(b)
      fire_scatter(b)
      b2 = (b + 2) % NSLOT

      @pl.when((sv >= 1) & (sv + 2 < ns_me))
      def _():
        wait_scatter(b2)
        fire_in(sv + 2, b2)

      @pl.when((sv < 1) & (sv + 2 < ns_me))
      def _():
        fire_in(sv + 2, b2)

    return carry

  lax.fori_loop(0, ns_me // NSLOT, body, 0)

  for b in range(NSLOT):
    wait_scatter(b)
  plsc.subcore_barrier()
  pltpu.sync_copy(accum.at[pl.ds(r0, RPT)], out.at[pl.ds(c * NP + r0, RPT)])


@functools.cache
def _sc_edge_call():
  mesh = plsc.VectorSubcoreMesh(
      core_axis_name="c", subcore_axis_name="s",
      num_cores=NSC, num_subcores=NTILE)
  return pl.kernel(
      _sc_edge_body,
      out_type=jax.ShapeDtypeStruct((NSC * NP, EMB), jnp.float32),
      mesh=mesh,
      scratch_types=(
          [pltpu.VMEM_SHARED((NP, EMB), jnp.float32)]
          + [pltpu.VMEM((SCW,), jnp.int32) for _ in range(NSLOT)]
          + [pltpu.VMEM((SCW,), jnp.int32) for _ in range(NSLOT)]
          + [pltpu.VMEM((SCW, EMB), jnp.float32) for _ in range(NSLOT)]
          + [pltpu.SemaphoreType.DMA for _ in range(3 * NSLOT)]
      ),
  )


def _sc_edge_pass(hl, srcr, dstr, ea):
  return _sc_edge_call()(hl, srcr, dstr, ea)


BLK = 2000
GRID = N // BLK


def _onehot(b_ref):
  return (b_ref[...] == lax.broadcasted_iota(jnp.int32, (BLK, B), 1)
          ).astype(jnp.float32)


def _segsum(oh, x):
  return lax.dot_general(oh, x, (((0,), (0,)), ((), ())),
                         preferred_element_type=jnp.float32)


def _vn_mlp(vt, w1_ref, b1_ref, w2_ref, b2_ref):
  u = jnp.maximum(
      jnp.dot(vt, w1_ref[...], preferred_element_type=jnp.float32)
      + b1_ref[...], 0.0)
  return jnp.maximum(
      jnp.dot(u, w2_ref[...], preferred_element_type=jnp.float32)
      + b2_ref[...], 0.0)


def _init_body(x_ref, b_ref, vne_ref, w1_ref, b1_ref, w2_ref, b2_ref,
               hl_ref, vn1_ref, seg_ref):
  i = pl.program_id(0)
  hl = x_ref[...] + vne_ref[...]
  hl_ref[...] = hl
  part = _segsum(_onehot(b_ref), hl)

  @pl.when(i == 0)
  def _():
    seg_ref[...] = part

  @pl.when(i > 0)
  def _():
    seg_ref[...] += part

  @pl.when(i == GRID - 1)
  def _():
    vt = seg_ref[...] + vne_ref[...]
    vn1_ref[...] = _vn_mlp(vt, w1_ref, b1_ref, w2_ref, b2_ref)


def _mid_body(hl_ref, aa_ref, ab_ref, b_ref, vn_ref, eps_ref,
              w1_ref, b1_ref, w2_ref, b2_ref,
              vw1_ref, vb1_ref, vw2_ref, vb2_ref,
              out_ref, vn2_ref, seg_ref):
  i = pl.program_id(0)
  z2 = eps_ref[0, 0] * hl_ref[...] + aa_ref[...] + ab_ref[...]
  u = jnp.maximum(
      jnp.dot(z2, w1_ref[...], preferred_element_type=jnp.float32)
      + b1_ref[...], 0.0)
  z = jnp.dot(u, w2_ref[...], preferred_element_type=jnp.float32) + b2_ref[...]
  h = jnp.maximum(z, 0.0)
  oh = _onehot(b_ref)
  hln = h + jnp.dot(oh, vn_ref[...], preferred_element_type=jnp.float32)
  out_ref[...] = hln
  part = _segsum(oh, hln)

  @pl.when(i == 0)
  def _():
    seg_ref[...] = part

  @pl.when(i > 0)
  def _():
    seg_ref[...] += part

  @pl.when(i == GRID - 1)
  def _():
    vt = seg_ref[...] + vn_ref[...]
    vn2_ref[...] = _vn_mlp(vt, vw1_ref, vb1_ref, vw2_ref, vb2_ref)


def _last_mid_body(hl_ref, aa_ref, ab_ref, b_ref, vn_ref, eps_ref,
                   w1_ref, b1_ref, w2_ref, b2_ref, out_ref):
  z2 = eps_ref[0, 0] * hl_ref[...] + aa_ref[...] + ab_ref[...]
  u = jnp.maximum(
      jnp.dot(z2, w1_ref[...], preferred_element_type=jnp.float32)
      + b1_ref[...], 0.0)
  z = jnp.dot(u, w2_ref[...], preferred_element_type=jnp.float32) + b2_ref[...]
  h = jnp.maximum(z, 0.0)
  oh = _onehot(b_ref)
  out_ref[...] = h + jnp.dot(oh, vn_ref[...],
                             preferred_element_type=jnp.float32)


def _final_body(hl_ref, aa_ref, ab_ref, b_ref, eps_ref,
                w1_ref, b1_ref, w2_ref, b2_ref, hw_ref, hb_ref,
                out_ref, pool_ref, cnt_ref):
  i = pl.program_id(0)
  z2 = eps_ref[0, 0] * hl_ref[...] + aa_ref[...] + ab_ref[...]
  u = jnp.maximum(
      jnp.dot(z2, w1_ref[...], preferred_element_type=jnp.float32)
      + b1_ref[...], 0.0)
  z = jnp.dot(u, w2_ref[...], preferred_element_type=jnp.float32) + b2_ref[...]
  oh = _onehot(b_ref)
  part = _segsum(oh, z)
  pcnt = _segsum(oh, jnp.ones((BLK, EMB), jnp.float32))

  @pl.when(i == 0)
  def _():
    pool_ref[...] = part
    cnt_ref[...] = pcnt

  @pl.when(i > 0)
  def _():
    pool_ref[...] += part
    cnt_ref[...] += pcnt

  @pl.when(i == GRID - 1)
  def _():
    hg = pool_ref[...] / jnp.maximum(cnt_ref[...], 1.0)
    out_ref[...] = (jnp.dot(hg, hw_ref[...],
                            preferred_element_type=jnp.float32)
                    + hb_ref[...])


def _blk(i):
  return (i, 0)


def _full(i):
  return (0, 0)


_ROW = pl.BlockSpec((BLK, EMB), _blk)
_ROWB = pl.BlockSpec((BLK, 1), _blk)


def _fullspec(shape):
  return pl.BlockSpec(shape, _full)


def _tc_init(x, batch2, vne, w1, b1, w2, b2):
  return pl.pallas_call(
      _init_body,
      grid=(GRID,),
      in_specs=[
          _ROW, _ROWB, _fullspec((1, EMB)),
          _fullspec((EMB, 2 * EMB)), _fullspec((1, 2 * EMB)),
          _fullspec((2 * EMB, EMB)), _fullspec((1, EMB)),
      ],
      out_specs=[_ROW, _fullspec((B, EMB))],
      out_shape=[
          jax.ShapeDtypeStruct((N, EMB), jnp.float32),
          jax.ShapeDtypeStruct((B, EMB), jnp.float32),
      ],
      scratch_shapes=[pltpu.VMEM((B, EMB), jnp.float32)],
  )(x, batch2, vne, w1, b1, w2, b2)


def _tc_mid(hl, aa, ab, batch2, vn, eps, w1, b1, w2, b2, vw1, vb1, vw2, vb2):
  return pl.pallas_call(
      _mid_body,
      grid=(GRID,),
      in_specs=[
          _ROW, _ROW, _ROW, _ROWB, _fullspec((B, EMB)), _fullspec((1, 1)),
          _fullspec((EMB, 2 * EMB)), _fullspec((1, 2 * EMB)),
          _fullspec((2 * EMB, EMB)), _fullspec((1, EMB)),
          _fullspec((EMB, 2 * EMB)), _fullspec((1, 2 * EMB)),
          _fullspec((2 * EMB, EMB)), _fullspec((1, EMB)),
      ],
      out_specs=[_ROW, _fullspec((B, EMB))],
      out_shape=[
          jax.ShapeDtypeStruct((N, EMB), jnp.float32),
          jax.ShapeDtypeStruct((B, EMB), jnp.float32),
      ],
      scratch_shapes=[pltpu.VMEM((B, EMB), jnp.float32)],
  )(hl, aa, ab, batch2, vn, eps, w1, b1, w2, b2, vw1, vb1, vw2, vb2)


def _tc_last_mid(hl, aa, ab, batch2, vn, eps, w1, b1, w2, b2):
  return pl.pallas_call(
      _last_mid_body,
      grid=(GRID,),
      in_specs=[
          _ROW, _ROW, _ROW, _ROWB, _fullspec((B, EMB)), _fullspec((1, 1)),
          _fullspec((EMB, 2 * EMB)), _fullspec((1, 2 * EMB)),
          _fullspec((2 * EMB, EMB)), _fullspec((1, EMB)),
      ],
      out_specs=_ROW,
      out_shape=jax.ShapeDtypeStruct((N, EMB), jnp.float32),
  )(hl, aa, ab, batch2, vn, eps, w1, b1, w2, b2)


def _tc_final(hl, aa, ab, batch2, eps, w1, b1, w2, b2, hw, hb):
  return pl.pallas_call(
      _final_body,
      grid=(GRID,),
      in_specs=[
          _ROW, _ROW, _ROW, _ROWB, _fullspec((1, 1)),
          _fullspec((EMB, 2 * EMB)), _fullspec((1, 2 * EMB)),
          _fullspec((2 * EMB, EMB)), _fullspec((1, EMB)),
          _fullspec((EMB, NC)), _fullspec((1, NC)),
      ],
      out_specs=_fullspec((B, NC)),
      out_shape=jax.ShapeDtypeStruct((B, NC), jnp.float32),
      scratch_shapes=[
          pltpu.VMEM((B, EMB), jnp.float32),
          pltpu.VMEM((B, EMB), jnp.float32),
      ],
  )(hl, aa, ab, batch2, eps, w1, b1, w2, b2, hw, hb)


_INV = 1.0 / jnp.sqrt(1.0 + 1e-5)


def kernel(x, edge_index, edge_attr, batch, gin_W1, gin_b1, gin_bn1_g,
           gin_bn1_b, gin_W2, gin_b2, eps, bn_g, bn_b, vn_emb, vn_W1, vn_b1,
           vn_bn1_g, vn_bn1_b, vn_W2, vn_b2, vn_bn2_g, vn_bn2_b, head_W,
           head_b):
  src = jnp.concatenate(
      [edge_index[0], jnp.zeros((EPADT - E,), jnp.int32)])
  dst = jnp.concatenate(
      [edge_index[1], jnp.full((EPADT - E,), NP - 1, jnp.int32)])
  s1 = gin_bn1_g * _INV
  w1f = gin_W1 * s1[:, None, :]
  b1f = (gin_b1 * s1 + gin_bn1_b)[:, None, :]
  s2 = bn_g * _INV
  w2f = gin_W2 * s2[:, None, :]
  b2f = (gin_b2 * s2 + bn_b)[:, None, :]
  t1 = vn_bn1_g * _INV
  vw1f = vn_W1 * t1[:, None, :]
  vb1f = (vn_b1 * t1 + vn_bn1_b)[:, None, :]
  t2 = vn_bn2_g * _INV
  vw2f = vn_W2 * t2[:, None, :]
  vb2f = (vn_b2 * t2 + vn_bn2_b)[:, None, :]
  eps1 = (1.0 + eps).reshape(NL, 1, 1)
  batch2 = batch.reshape(N, 1)
  vne = vn_emb.reshape(1, EMB)
  hb2 = head_b.reshape(1, NC)

  hl, vn = _tc_init(x, batch2, vne, vw1f[0], vb1f[0], vw2f[0], vb2f[0])
  for l in range(NL - 1):
    part = _sc_edge_pass(hl, src, dst, edge_attr)
    aa = part[:N]
    ab = part[NP:NP + N]
    if l < NL - 2:
      hl, vn = _tc_mid(hl, aa, ab, batch2, vn, eps1[l],
                       w1f[l], b1f[l], w2f[l], b2f[l],
                       vw1f[l + 1], vb1f[l + 1], vw2f[l + 1], vb2f[l + 1])
    else:
      hl = _tc_last_mid(hl, aa, ab, batch2, vn, eps1[l],
                        w1f[l], b1f[l], w2f[l], b2f[l])
  part = _sc_edge_pass(hl, src, dst, edge_attr)
  return _tc_final(hl, part[:N], part[NP:NP + N], batch2, eps1[NL - 1],
                   w1f[NL - 1], b1f[NL - 1], w2f[NL - 1], b2f[NL - 1],
                   head_W, hb2)
